# baseline (speedup 1.0000x reference)
import functools

import numpy as np
import jax
import jax.numpy as jnp
from jax.experimental import pallas as pl
from jax.experimental.pallas import tpu as pltpu


def _fused_head_kernel(ts_ref, vs_ref, emb_ref, w1_ref, b1_ref, u_ref,
                       w2r_ref, wl_ref, bl_ref, sc_ref, out_ref,
                       *, H, K, TM, SE, NP1, S, L, T, neg=0.1):
    i = pl.program_id(1)

    a_min = (i * TM) // NP1
    st = jnp.minimum(jnp.maximum(a_min - (K - 1), 0), S - SE)
    st = (st // 32) * 32

    e_slab = emb_ref[pl.ds(st, SE), :].astype(jnp.bfloat16)

    rows = jax.lax.broadcasted_iota(jnp.int32, (TM, SE), 0)
    cols = jax.lax.broadcasted_iota(jnp.int32, (TM, SE), 1)
    aq = jnp.minimum((i * TM + rows) // NP1, S - 1)
    d = aq - (st + cols)
    vmask = ((d >= 0) & (d < K)).astype(jnp.float32)

    onehot0 = (d == 0).astype(jnp.float32)
    pv = jnp.dot(onehot0, vs_ref[...],
                 preferred_element_type=jnp.float32)
    t_a = pv[:, 0:1]
    t_n = pv[:, 1:2]
    npm_a = pv[:, 2:3]
    npm_n = pv[:, 3:4]
    dtn = pv[:, 4:5]
    evi = pv[:, 5:6].astype(jnp.int32)

    p_col = i * TM + jax.lax.broadcasted_iota(jnp.int32, (TM, 1), 0)
    r_col = p_col % NP1
    a_col = jnp.minimum(p_col // NP1, S - 1)
    u_col = jnp.zeros((TM, 1), jnp.float32)
    for j in range(1, NP1):
        u_col = jnp.where(r_col == j, u_ref[0, j], u_col)

    tau = t_a + u_col * (t_n - t_a)
    dt = tau - ts_ref[...]
    dtm = dt * vmask

    wint = (((r_col > 0) & (a_col <= S - 2)).astype(jnp.float32)
            * dtn * npm_n * (1.0 / (NP1 - 1)))
    wev = (((r_col == 0) & (p_col < L)).astype(jnp.float32)
           * npm_a)

    C = e_slab.shape[1]
    g = jnp.dot(e_slab, w2r_ref[...],
                preferred_element_type=jnp.float32)
    g = g.astype(jnp.bfloat16)

    conv = jnp.zeros((TM, C), jnp.float32)
    for h in range(H):
        pre = dtm * w1_ref[0, h] + b1_ref[0, h]
        act = jnp.where(pre >= 0, pre, neg * pre) * vmask
        conv += jnp.dot(act.astype(jnp.bfloat16), g[:, h * C:(h + 1) * C],
                        preferred_element_type=jnp.float32)
    conv += jnp.dot(vmask.astype(jnp.bfloat16), g[:, H * C:(H + 1) * C],
                    preferred_element_type=jnp.float32)
    conv = jnp.where(conv >= 0, conv, neg * conv)

    z = jnp.dot(conv.astype(jnp.bfloat16), wl_ref[...],
                preferred_element_type=jnp.float32) + bl_ref[...]
    sp = jnp.maximum(z, 0.0) + jnp.log1p(jnp.exp(-jnp.abs(z)))
    lam = sp * sc_ref[...]

    intg = jnp.sum(lam * wint)

    tcols = jax.lax.broadcasted_iota(jnp.int32, (TM, T), 1)
    sel = jnp.sum(jnp.where(tcols == evi, lam, 0.0),
                  axis=1, keepdims=True)
    evl = -jnp.log(sel + 1e-8) * wev

    part = intg + jnp.sum(evl)

    @pl.when(i == 0)
    def _():
        out_ref[...] = jnp.zeros_like(out_ref)
    out_ref[...] += part


def kernel(w1, b1, w2, b2, wl, bl, softplus_params, times, embeddings,
           events, non_pad_mask, uniform_sample):
    B, S = times.shape
    C = embeddings.shape[-1]
    T = wl.shape[-1]
    H = w1.shape[-1]
    K = min(7, S)
    N = int(uniform_sample.shape[0])
    NP1 = N + 1
    L = (S - 1) * NP1 + 1

    TM = 256
    n_tiles = -(-L // TM)
    L_pad = TM * n_tiles
    SE = 96

    f32 = jnp.float32
    bf16 = jnp.bfloat16

    mx = jnp.max(times, axis=1)
    mx = jnp.where(mx > 0, mx, 1.0)
    ts_s = times / mx[:, None]

    ts_next = jnp.concatenate([ts_s[:, 1:], ts_s[:, -1:]], axis=1)
    tn = jnp.concatenate([times[:, 1:], times[:, -1:]], axis=1)
    npm_next = jnp.concatenate(
        [non_pad_mask[:, 1:], non_pad_mask[:, -1:]], axis=1)
    ev_idx = jnp.where(events - 1 == -1, 0, events - 1).astype(f32)
    zeros = jnp.zeros((B, S), f32)
    v_full = jnp.stack(
        [ts_s, ts_next, non_pad_mask, npm_next, tn - times, ev_idx,
         zeros, zeros], axis=-1)

    starts = np.minimum(np.maximum(np.arange(n_tiles) * TM // NP1 - (K - 1),
                                   0), S - SE)
    starts = (starts // 32) * 32
    t_slab = jnp.stack(
        [jax.lax.slice(ts_s, (0, int(st)), (B, int(st) + SE))
         for st in starts], axis=1)[:, :, None, :]
    v_slab = jnp.stack(
        [jax.lax.slice(v_full, (0, int(st), 0), (B, int(st) + SE, 8))
         for st in starts], axis=1)

    w2e = jnp.concatenate([w2, b2], axis=0)
    w2r = (w2e.reshape(H + 1, C, C).transpose(1, 0, 2)
           .reshape(C, (H + 1) * C).astype(bf16))
    wl_bf = wl.astype(bf16)
    sc = jnp.exp(softplus_params)[0]
    w1p = jnp.zeros((1, 128), f32).at[0, :H].set(w1[0])
    b1p = jnp.zeros((1, 128), f32).at[0, :H].set(b1[0])
    up = jnp.zeros((1, 128), f32).at[0, 1:NP1].set(uniform_sample)

    body = functools.partial(_fused_head_kernel, H=H, K=K, TM=TM, SE=SE,
                             NP1=NP1, S=S, L=L, T=T)
    out = pl.pallas_call(
        body,
        out_shape=jax.ShapeDtypeStruct((B, 1, 128), f32),
        grid=(B, n_tiles),
        in_specs=[
            pl.BlockSpec((None, None, 1, SE),
                         lambda b, i: (b, i, 0, 0)),
            pl.BlockSpec((None, None, SE, 8),
                         lambda b, i: (b, i, 0, 0)),
            pl.BlockSpec((None, S, C), lambda b, i: (b, 0, 0)),
            pl.BlockSpec((1, 128), lambda b, i: (0, 0)),
            pl.BlockSpec((1, 128), lambda b, i: (0, 0)),
            pl.BlockSpec((1, 128), lambda b, i: (0, 0)),
            pl.BlockSpec((C, (H + 1) * C), lambda b, i: (0, 0)),
            pl.BlockSpec((C, T), lambda b, i: (0, 0)),
            pl.BlockSpec((1, T), lambda b, i: (0, 0)),
            pl.BlockSpec((1, T), lambda b, i: (0, 0)),
        ],
        out_specs=pl.BlockSpec((None, 1, 128), lambda b, i: (b, 0, 0)),
        compiler_params=pltpu.CompilerParams(
            dimension_semantics=("parallel", "arbitrary")),
    )(t_slab, v_slab, embeddings, w1p, b1p, up, w2r, wl_bf, bl, sc)

    return jnp.sum(out[:, 0, 0])


# speedup vs baseline: 1.2076x; 1.2076x over previous
import functools

import numpy as np
import jax
import jax.numpy as jnp
from jax.experimental import pallas as pl
from jax.experimental.pallas import tpu as pltpu


def _fused_head_kernel(ts_ref, vs_ref, emb_ref, w1_ref, b1_ref, u_ref,
                       w2r_ref, wl_ref, bl_ref, sc_ref, out_ref,
                       *, H, K, TM, SE, NP1, S, L, T, neg=0.1):
    i = pl.program_id(1)

    a_min = (i * TM) // NP1
    st = jnp.minimum(jnp.maximum(a_min - (K - 1), 0), S - SE)
    st = (st // 32) * 32

    e_slab = emb_ref[pl.ds(st, SE), :].astype(jnp.bfloat16)

    rows = jax.lax.broadcasted_iota(jnp.int32, (TM, SE), 0)
    cols = jax.lax.broadcasted_iota(jnp.int32, (TM, SE), 1)
    aq = jnp.minimum((i * TM + rows) // NP1, S - 1)
    d = aq - (st + cols)
    vmask = ((d >= 0) & (d < K)).astype(jnp.float32)

    onehot0 = (d == 0).astype(jnp.float32)
    pv = jnp.dot(onehot0, vs_ref[...],
                 preferred_element_type=jnp.float32)
    t_a = pv[:, 0:1]
    t_n = pv[:, 1:2]
    npm_a = pv[:, 2:3]
    npm_n = pv[:, 3:4]
    dtn = pv[:, 4:5]
    evi = pv[:, 5:6].astype(jnp.int32)

    p_col = i * TM + jax.lax.broadcasted_iota(jnp.int32, (TM, 1), 0)
    r_col = p_col % NP1
    a_col = jnp.minimum(p_col // NP1, S - 1)
    u_col = jnp.zeros((TM, 1), jnp.float32)
    for j in range(1, NP1):
        u_col = jnp.where(r_col == j, u_ref[0, j], u_col)

    tau = t_a + u_col * (t_n - t_a)
    dt = tau - ts_ref[...]
    dtm = dt * vmask

    wint = (((r_col > 0) & (a_col <= S - 2)).astype(jnp.float32)
            * dtn * npm_n * (1.0 / (NP1 - 1)))
    wev = (((r_col == 0) & (p_col < L)).astype(jnp.float32)
           * npm_a)

    C = e_slab.shape[1]
    g = jnp.dot(e_slab, w2r_ref[...],
                preferred_element_type=jnp.float32)
    g = g.astype(jnp.bfloat16)

    conv = jnp.zeros((TM, C), jnp.float32)
    for h in range(H):
        pre = dtm * w1_ref[0, h] + b1_ref[0, h]
        act = jnp.where(pre >= 0, pre, neg * pre) * vmask
        conv += jnp.dot(act.astype(jnp.bfloat16), g[:, h * C:(h + 1) * C],
                        preferred_element_type=jnp.float32)
    conv += jnp.dot(vmask.astype(jnp.bfloat16), g[:, H * C:(H + 1) * C],
                    preferred_element_type=jnp.float32)
    conv = jnp.where(conv >= 0, conv, neg * conv)

    z = jnp.dot(conv.astype(jnp.bfloat16), wl_ref[...],
                preferred_element_type=jnp.float32) + bl_ref[...]
    sp = jnp.maximum(z, 0.0) + jnp.log1p(jnp.exp(-jnp.abs(z)))
    lam = sp * sc_ref[...]

    intg = jnp.sum(lam * wint)

    tcols = jax.lax.broadcasted_iota(jnp.int32, (TM, T), 1)
    sel = jnp.sum(jnp.where(tcols == evi, lam, 0.0),
                  axis=1, keepdims=True)
    evl = -jnp.log(sel + 1e-8) * wev

    part = intg + jnp.sum(evl)

    @pl.when(i == 0)
    def _():
        out_ref[...] = jnp.zeros_like(out_ref)
    out_ref[...] += part


def kernel(w1, b1, w2, b2, wl, bl, softplus_params, times, embeddings,
           events, non_pad_mask, uniform_sample):
    B, S = times.shape
    C = embeddings.shape[-1]
    T = wl.shape[-1]
    H = w1.shape[-1]
    K = min(7, S)
    N = int(uniform_sample.shape[0])
    NP1 = N + 1
    L = (S - 1) * NP1 + 1

    TM = 1024
    n_tiles = -(-L // TM)
    L_pad = TM * n_tiles
    SE = 256

    f32 = jnp.float32
    bf16 = jnp.bfloat16

    mx = jnp.max(times, axis=1)
    mx = jnp.where(mx > 0, mx, 1.0)
    ts_s = times / mx[:, None]

    ts_next = jnp.concatenate([ts_s[:, 1:], ts_s[:, -1:]], axis=1)
    tn = jnp.concatenate([times[:, 1:], times[:, -1:]], axis=1)
    npm_next = jnp.concatenate(
        [non_pad_mask[:, 1:], non_pad_mask[:, -1:]], axis=1)
    ev_idx = jnp.where(events - 1 == -1, 0, events - 1).astype(f32)
    zeros = jnp.zeros((B, S), f32)
    v_full = jnp.stack(
        [ts_s, ts_next, non_pad_mask, npm_next, tn - times, ev_idx,
         zeros, zeros], axis=-1)

    starts = np.minimum(np.maximum(np.arange(n_tiles) * TM // NP1 - (K - 1),
                                   0), S - SE)
    starts = (starts // 32) * 32
    t_slab = jnp.stack(
        [jax.lax.slice(ts_s, (0, int(st)), (B, int(st) + SE))
         for st in starts], axis=1)[:, :, None, :]
    v_slab = jnp.stack(
        [jax.lax.slice(v_full, (0, int(st), 0), (B, int(st) + SE, 8))
         for st in starts], axis=1)

    w2e = jnp.concatenate([w2, b2], axis=0)
    w2r = (w2e.reshape(H + 1, C, C).transpose(1, 0, 2)
           .reshape(C, (H + 1) * C).astype(bf16))
    wl_bf = wl.astype(bf16)
    sc = jnp.exp(softplus_params)[0]
    w1p = jnp.zeros((1, 128), f32).at[0, :H].set(w1[0])
    b1p = jnp.zeros((1, 128), f32).at[0, :H].set(b1[0])
    up = jnp.zeros((1, 128), f32).at[0, 1:NP1].set(uniform_sample)

    body = functools.partial(_fused_head_kernel, H=H, K=K, TM=TM, SE=SE,
                             NP1=NP1, S=S, L=L, T=T)
    out = pl.pallas_call(
        body,
        out_shape=jax.ShapeDtypeStruct((B, 1, 128), f32),
        grid=(B, n_tiles),
        in_specs=[
            pl.BlockSpec((None, None, 1, SE),
                         lambda b, i: (b, i, 0, 0)),
            pl.BlockSpec((None, None, SE, 8),
                         lambda b, i: (b, i, 0, 0)),
            pl.BlockSpec((None, S, C), lambda b, i: (b, 0, 0)),
            pl.BlockSpec((1, 128), lambda b, i: (0, 0)),
            pl.BlockSpec((1, 128), lambda b, i: (0, 0)),
            pl.BlockSpec((1, 128), lambda b, i: (0, 0)),
            pl.BlockSpec((C, (H + 1) * C), lambda b, i: (0, 0)),
            pl.BlockSpec((C, T), lambda b, i: (0, 0)),
            pl.BlockSpec((1, T), lambda b, i: (0, 0)),
            pl.BlockSpec((1, T), lambda b, i: (0, 0)),
        ],
        out_specs=pl.BlockSpec((None, 1, 128), lambda b, i: (b, 0, 0)),
        compiler_params=pltpu.CompilerParams(
            dimension_semantics=("parallel", "arbitrary")),
    )(t_slab, v_slab, embeddings, w1p, b1p, up, w2r, wl_bf, bl, sc)

    return jnp.sum(out[:, 0, 0])


# speedup vs baseline: 1.2734x; 1.0545x over previous
import functools

import numpy as np
import jax
import jax.numpy as jnp
from jax.experimental import pallas as pl
from jax.experimental.pallas import tpu as pltpu


def _fused_head_kernel(ts_ref, vs_ref, emb_ref, w1_ref, b1_ref, u_ref,
                       w2r_ref, wl_ref, bl_ref, sc_ref, out_ref,
                       *, H, K, TM, SE, NP1, S, L, T, neg=0.1):
    i = pl.program_id(1)

    a_min = (i * TM) // NP1
    st = jnp.minimum(jnp.maximum(a_min - (K - 1), 0), S - SE)
    st = (st // 8) * 8

    e_slab = emb_ref[pl.ds(st, SE), :].astype(jnp.bfloat16)

    rows = jax.lax.broadcasted_iota(jnp.int32, (TM, SE), 0)
    cols = jax.lax.broadcasted_iota(jnp.int32, (TM, SE), 1)
    aq = jnp.minimum((i * TM + rows) // NP1, S - 1)
    d = aq - (st + cols)
    vmask = ((d >= 0) & (d < K)).astype(jnp.float32)

    onehot0 = (d == 0).astype(jnp.float32)
    pv = jnp.dot(onehot0, vs_ref[...],
                 preferred_element_type=jnp.float32)
    t_a = pv[:, 0:1]
    t_n = pv[:, 1:2]
    npm_a = pv[:, 2:3]
    npm_n = pv[:, 3:4]
    dtn = pv[:, 4:5]
    evi = pv[:, 5:6].astype(jnp.int32)

    p_col = i * TM + jax.lax.broadcasted_iota(jnp.int32, (TM, 1), 0)
    r_col = p_col % NP1
    a_col = jnp.minimum(p_col // NP1, S - 1)
    u_col = jnp.zeros((TM, 1), jnp.float32)
    for j in range(1, NP1):
        u_col = jnp.where(r_col == j, u_ref[0, j], u_col)

    tau = t_a + u_col * (t_n - t_a)
    dt = tau - ts_ref[...]
    dtm = dt * vmask

    wint = (((r_col > 0) & (a_col <= S - 2)).astype(jnp.float32)
            * dtn * npm_n * (1.0 / (NP1 - 1)))
    wev = (((r_col == 0) & (p_col < L)).astype(jnp.float32)
           * npm_a)

    C = e_slab.shape[1]
    g = jnp.dot(e_slab, w2r_ref[...],
                preferred_element_type=jnp.float32)
    g = g.astype(jnp.bfloat16)

    conv = jnp.zeros((TM, C), jnp.float32)
    for h in range(H):
        pre = dtm * w1_ref[0, h] + b1_ref[0, h]
        act = jnp.maximum(pre, neg * pre) * vmask
        conv += jnp.dot(act.astype(jnp.bfloat16), g[:, h * C:(h + 1) * C],
                        preferred_element_type=jnp.float32)
    conv += jnp.dot(vmask.astype(jnp.bfloat16), g[:, H * C:(H + 1) * C],
                    preferred_element_type=jnp.float32)
    conv = jnp.where(conv >= 0, conv, neg * conv)

    z = jnp.dot(conv.astype(jnp.bfloat16), wl_ref[...],
                preferred_element_type=jnp.float32) + bl_ref[...]
    sp = jnp.maximum(z, 0.0) + jnp.log1p(jnp.exp(-jnp.abs(z)))
    lam = sp * sc_ref[...]

    intg = jnp.sum(lam * wint)

    tcols = jax.lax.broadcasted_iota(jnp.int32, (TM, T), 1)
    sel = jnp.sum(jnp.where(tcols == evi, lam, 0.0),
                  axis=1, keepdims=True)
    evl = -jnp.log(sel + 1e-8) * wev

    part = intg + jnp.sum(evl)

    @pl.when(i == 0)
    def _():
        out_ref[...] = jnp.zeros_like(out_ref)
    out_ref[...] += part


def kernel(w1, b1, w2, b2, wl, bl, softplus_params, times, embeddings,
           events, non_pad_mask, uniform_sample):
    B, S = times.shape
    C = embeddings.shape[-1]
    T = wl.shape[-1]
    H = w1.shape[-1]
    K = min(7, S)
    N = int(uniform_sample.shape[0])
    NP1 = N + 1
    L = (S - 1) * NP1 + 1

    TM = 1024
    n_tiles = -(-L // TM)
    L_pad = TM * n_tiles
    SE = 224

    f32 = jnp.float32
    bf16 = jnp.bfloat16

    mx = jnp.max(times, axis=1)
    mx = jnp.where(mx > 0, mx, 1.0)
    ts_s = times / mx[:, None]

    ts_next = jnp.concatenate([ts_s[:, 1:], ts_s[:, -1:]], axis=1)
    tn = jnp.concatenate([times[:, 1:], times[:, -1:]], axis=1)
    npm_next = jnp.concatenate(
        [non_pad_mask[:, 1:], non_pad_mask[:, -1:]], axis=1)
    ev_idx = jnp.where(events - 1 == -1, 0, events - 1).astype(f32)
    zeros = jnp.zeros((B, S), f32)
    v_full = jnp.stack(
        [ts_s, ts_next, non_pad_mask, npm_next, tn - times, ev_idx,
         zeros, zeros], axis=-1)

    starts = np.minimum(np.maximum(np.arange(n_tiles) * TM // NP1 - (K - 1),
                                   0), S - SE)
    starts = (starts // 8) * 8
    t_slab = jnp.stack(
        [jax.lax.slice(ts_s, (0, int(st)), (B, int(st) + SE))
         for st in starts], axis=1)[:, :, None, :]
    v_slab = jnp.stack(
        [jax.lax.slice(v_full, (0, int(st), 0), (B, int(st) + SE, 8))
         for st in starts], axis=1)

    w2e = jnp.concatenate([w2, b2], axis=0)
    w2r = (w2e.reshape(H + 1, C, C).transpose(1, 0, 2)
           .reshape(C, (H + 1) * C).astype(bf16))
    wl_bf = wl.astype(bf16)
    sc = jnp.exp(softplus_params)[0]
    w1p = jnp.zeros((1, 128), f32).at[0, :H].set(w1[0])
    b1p = jnp.zeros((1, 128), f32).at[0, :H].set(b1[0])
    up = jnp.zeros((1, 128), f32).at[0, 1:NP1].set(uniform_sample)

    body = functools.partial(_fused_head_kernel, H=H, K=K, TM=TM, SE=SE,
                             NP1=NP1, S=S, L=L, T=T)
    out = pl.pallas_call(
        body,
        out_shape=jax.ShapeDtypeStruct((B, 1, 128), f32),
        grid=(B, n_tiles),
        in_specs=[
            pl.BlockSpec((None, None, 1, SE),
                         lambda b, i: (b, i, 0, 0)),
            pl.BlockSpec((None, None, SE, 8),
                         lambda b, i: (b, i, 0, 0)),
            pl.BlockSpec((None, S, C), lambda b, i: (b, 0, 0)),
            pl.BlockSpec((1, 128), lambda b, i: (0, 0)),
            pl.BlockSpec((1, 128), lambda b, i: (0, 0)),
            pl.BlockSpec((1, 128), lambda b, i: (0, 0)),
            pl.BlockSpec((C, (H + 1) * C), lambda b, i: (0, 0)),
            pl.BlockSpec((C, T), lambda b, i: (0, 0)),
            pl.BlockSpec((1, T), lambda b, i: (0, 0)),
            pl.BlockSpec((1, T), lambda b, i: (0, 0)),
        ],
        out_specs=pl.BlockSpec((None, 1, 128), lambda b, i: (b, 0, 0)),
        compiler_params=pltpu.CompilerParams(
            dimension_semantics=("parallel", "arbitrary")),
    )(t_slab, v_slab, embeddings, w1p, b1p, up, w2r, wl_bf, bl, sc)

    return jnp.sum(out[:, 0, 0])


# speedup vs baseline: 1.4288x; 1.1220x over previous
import functools

import numpy as np
import jax
import jax.numpy as jnp
from jax.experimental import pallas as pl
from jax.experimental.pallas import tpu as pltpu


def _fused_head_kernel(ts_ref, vs_ref, emb_ref, w1_ref, b1_ref, u_ref,
                       w2r_ref, wl_ref, bl_ref, sc_ref, out_ref,
                       *, H, K, TM, SE, NP1, S, L, T, neg=0.1):
    i = pl.program_id(1)

    a_min = (i * TM) // NP1
    st = jnp.minimum(jnp.maximum(a_min - (K - 1), 0), S - SE)
    st = (st // 8) * 8

    e_slab = emb_ref[pl.ds(st, SE), :].astype(jnp.bfloat16)

    rows = jax.lax.broadcasted_iota(jnp.int32, (TM, SE), 0)
    cols = jax.lax.broadcasted_iota(jnp.int32, (TM, SE), 1)
    aq = jnp.minimum((i * TM + rows) // NP1, S - 1)
    d = aq - (st + cols)
    vmask = ((d >= 0) & (d < K)).astype(jnp.float32)

    onehot0 = (d == 0).astype(jnp.float32)
    pv = jnp.dot(onehot0, vs_ref[...],
                 preferred_element_type=jnp.float32)
    t_a = pv[:, 0:1]
    t_n = pv[:, 1:2]
    npm_a = pv[:, 2:3]
    npm_n = pv[:, 3:4]
    dtn = pv[:, 4:5]
    evi = pv[:, 5:6].astype(jnp.int32)

    p_col = i * TM + jax.lax.broadcasted_iota(jnp.int32, (TM, 1), 0)
    r_col = p_col % NP1
    a_col = jnp.minimum(p_col // NP1, S - 1)
    u_col = jnp.zeros((TM, 1), jnp.float32)
    for j in range(1, NP1):
        u_col = jnp.where(r_col == j, u_ref[0, j], u_col)

    tau = t_a + u_col * (t_n - t_a)
    dt = tau - ts_ref[...]
    vmask_bf = vmask.astype(jnp.bfloat16)
    dtm = (dt * vmask).astype(jnp.bfloat16)

    wint = (((r_col > 0) & (a_col <= S - 2)).astype(jnp.float32)
            * dtn * npm_n * (1.0 / (NP1 - 1)))
    wev = (((r_col == 0) & (p_col < L)).astype(jnp.float32)
           * npm_a)

    C = e_slab.shape[1]
    g = jnp.dot(e_slab, w2r_ref[...],
                preferred_element_type=jnp.float32)
    g = g.astype(jnp.bfloat16)

    negb = jnp.bfloat16(neg)
    conv = jnp.zeros((TM, C), jnp.float32)
    for h in range(H):
        pre = dtm * w1_ref[0, h].astype(jnp.bfloat16) \
            + b1_ref[0, h].astype(jnp.bfloat16)
        act = jnp.maximum(pre, negb * pre) * vmask_bf
        conv += jnp.dot(act, g[:, h * C:(h + 1) * C],
                        preferred_element_type=jnp.float32)
    conv += jnp.dot(vmask_bf, g[:, H * C:(H + 1) * C],
                    preferred_element_type=jnp.float32)
    conv = jnp.where(conv >= 0, conv, neg * conv)

    z = jnp.dot(conv.astype(jnp.bfloat16), wl_ref[...],
                preferred_element_type=jnp.float32) + bl_ref[...]
    sp = jnp.maximum(z, 0.0) + jnp.log1p(jnp.exp(-jnp.abs(z)))
    lam = sp * sc_ref[...]

    intg = jnp.sum(lam * wint)

    tcols = jax.lax.broadcasted_iota(jnp.int32, (TM, T), 1)
    sel = jnp.sum(jnp.where(tcols == evi, lam, 0.0),
                  axis=1, keepdims=True)
    evl = -jnp.log(sel + 1e-8) * wev

    part = intg + jnp.sum(evl)

    @pl.when(i == 0)
    def _():
        out_ref[...] = jnp.zeros_like(out_ref)
    out_ref[...] += part


def kernel(w1, b1, w2, b2, wl, bl, softplus_params, times, embeddings,
           events, non_pad_mask, uniform_sample):
    B, S = times.shape
    C = embeddings.shape[-1]
    T = wl.shape[-1]
    H = w1.shape[-1]
    K = min(7, S)
    N = int(uniform_sample.shape[0])
    NP1 = N + 1
    L = (S - 1) * NP1 + 1

    TM = 1024
    n_tiles = -(-L // TM)
    L_pad = TM * n_tiles
    SE = 224

    f32 = jnp.float32
    bf16 = jnp.bfloat16

    mx = jnp.max(times, axis=1)
    mx = jnp.where(mx > 0, mx, 1.0)
    ts_s = times / mx[:, None]

    ts_next = jnp.concatenate([ts_s[:, 1:], ts_s[:, -1:]], axis=1)
    tn = jnp.concatenate([times[:, 1:], times[:, -1:]], axis=1)
    npm_next = jnp.concatenate(
        [non_pad_mask[:, 1:], non_pad_mask[:, -1:]], axis=1)
    ev_idx = jnp.where(events - 1 == -1, 0, events - 1).astype(f32)
    zeros = jnp.zeros((B, S), f32)
    v_full = jnp.stack(
        [ts_s, ts_next, non_pad_mask, npm_next, tn - times, ev_idx,
         zeros, zeros], axis=-1)

    starts = np.minimum(np.maximum(np.arange(n_tiles) * TM // NP1 - (K - 1),
                                   0), S - SE)
    starts = (starts // 8) * 8
    t_slab = jnp.stack(
        [jax.lax.slice(ts_s, (0, int(st)), (B, int(st) + SE))
         for st in starts], axis=1)[:, :, None, :]
    v_slab = jnp.stack(
        [jax.lax.slice(v_full, (0, int(st), 0), (B, int(st) + SE, 8))
         for st in starts], axis=1)

    w2e = jnp.concatenate([w2, b2], axis=0)
    w2r = (w2e.reshape(H + 1, C, C).transpose(1, 0, 2)
           .reshape(C, (H + 1) * C).astype(bf16))
    wl_bf = wl.astype(bf16)
    sc = jnp.exp(softplus_params)[0]
    w1p = jnp.zeros((1, 128), f32).at[0, :H].set(w1[0])
    b1p = jnp.zeros((1, 128), f32).at[0, :H].set(b1[0])
    up = jnp.zeros((1, 128), f32).at[0, 1:NP1].set(uniform_sample)

    body = functools.partial(_fused_head_kernel, H=H, K=K, TM=TM, SE=SE,
                             NP1=NP1, S=S, L=L, T=T)
    out = pl.pallas_call(
        body,
        out_shape=jax.ShapeDtypeStruct((B, 1, 128), f32),
        grid=(B, n_tiles),
        in_specs=[
            pl.BlockSpec((None, None, 1, SE),
                         lambda b, i: (b, i, 0, 0)),
            pl.BlockSpec((None, None, SE, 8),
                         lambda b, i: (b, i, 0, 0)),
            pl.BlockSpec((None, S, C), lambda b, i: (b, 0, 0)),
            pl.BlockSpec((1, 128), lambda b, i: (0, 0)),
            pl.BlockSpec((1, 128), lambda b, i: (0, 0)),
            pl.BlockSpec((1, 128), lambda b, i: (0, 0)),
            pl.BlockSpec((C, (H + 1) * C), lambda b, i: (0, 0)),
            pl.BlockSpec((C, T), lambda b, i: (0, 0)),
            pl.BlockSpec((1, T), lambda b, i: (0, 0)),
            pl.BlockSpec((1, T), lambda b, i: (0, 0)),
        ],
        out_specs=pl.BlockSpec((None, 1, 128), lambda b, i: (b, 0, 0)),
        compiler_params=pltpu.CompilerParams(
            dimension_semantics=("parallel", "arbitrary")),
    )(t_slab, v_slab, embeddings, w1p, b1p, up, w2r, wl_bf, bl, sc)

    return jnp.sum(out[:, 0, 0])


# speedup vs baseline: 1.7622x; 1.2333x over previous
import functools

import numpy as np
import jax
import jax.numpy as jnp
from jax.experimental import pallas as pl
from jax.experimental.pallas import tpu as pltpu


def _fused_head_kernel(ts_ref, vs_ref, emb_ref, w1_ref, b1_ref, u_ref,
                       w2r_ref, wl_ref, bl_ref, sc_ref, out_ref,
                       *, H, K, TM, SE, NP1, S, L, T, neg=0.1):
    i = pl.program_id(1)

    a_min = (i * TM) // NP1
    st = jnp.minimum(jnp.maximum(a_min - (K - 1), 0), S - SE)
    st = (st // 8) * 8

    e_slab = emb_ref[pl.ds(st, SE), :].astype(jnp.bfloat16)

    rows = jax.lax.broadcasted_iota(jnp.int32, (TM, SE), 0)
    cols = jax.lax.broadcasted_iota(jnp.int32, (TM, SE), 1)
    aq = jnp.minimum((i * TM + rows) // NP1, S - 1)
    d = aq - (st + cols)
    vmask = ((d >= 0) & (d < K)).astype(jnp.float32)

    onehot0 = (d == 0).astype(jnp.float32)
    pv = jax.lax.dot_general(
        onehot0, vs_ref[...], (((1,), (1,)), ((), ())),
        preferred_element_type=jnp.float32)
    t_a = pv[:, 0:1]
    t_n = pv[:, 1:2]
    npm_a = pv[:, 2:3]
    npm_n = pv[:, 3:4]
    dtn = pv[:, 4:5]
    evi = pv[:, 5:6].astype(jnp.int32)

    p_col = i * TM + jax.lax.broadcasted_iota(jnp.int32, (TM, 1), 0)
    r_col = p_col % NP1
    a_col = jnp.minimum(p_col // NP1, S - 1)
    u_col = jnp.zeros((TM, 1), jnp.float32)
    for j in range(1, NP1):
        u_col = jnp.where(r_col == j, u_ref[0, j], u_col)

    tau = t_a + u_col * (t_n - t_a)
    dt = tau - ts_ref[...]
    vmask_bf = vmask.astype(jnp.bfloat16)
    dtm = (dt * vmask).astype(jnp.bfloat16)

    wint = (((r_col > 0) & (a_col <= S - 2)).astype(jnp.float32)
            * dtn * npm_n * (1.0 / (NP1 - 1)))
    wev = (((r_col == 0) & (p_col < L)).astype(jnp.float32)
           * npm_a)

    C = e_slab.shape[1]
    g = jnp.dot(e_slab, w2r_ref[...],
                preferred_element_type=jnp.float32)
    g = g.astype(jnp.bfloat16)

    negb = jnp.bfloat16(neg)
    conv = jnp.zeros((TM, C), jnp.float32)
    for h in range(H):
        pre = dtm * w1_ref[0, h].astype(jnp.bfloat16) \
            + b1_ref[0, h].astype(jnp.bfloat16)
        act = jnp.maximum(pre, negb * pre) * vmask_bf
        conv += jnp.dot(act, g[:, h * C:(h + 1) * C],
                        preferred_element_type=jnp.float32)
    conv += jnp.dot(vmask_bf, g[:, H * C:(H + 1) * C],
                    preferred_element_type=jnp.float32)
    conv = jnp.where(conv >= 0, conv, neg * conv)

    z = jnp.dot(conv.astype(jnp.bfloat16), wl_ref[...],
                preferred_element_type=jnp.float32) + bl_ref[...]
    sp = jnp.maximum(z, 0.0) + jnp.log1p(jnp.exp(-jnp.abs(z)))
    lam = sp * sc_ref[...]

    intg = jnp.sum(lam * wint)

    tcols = jax.lax.broadcasted_iota(jnp.int32, (TM, T), 1)
    sel = jnp.sum(jnp.where(tcols == evi, lam, 0.0),
                  axis=1, keepdims=True)
    evl = -jnp.log(sel + 1e-8) * wev

    part = intg + jnp.sum(evl)

    @pl.when(i == 0)
    def _():
        out_ref[...] = jnp.zeros_like(out_ref)
    out_ref[...] += part


def kernel(w1, b1, w2, b2, wl, bl, softplus_params, times, embeddings,
           events, non_pad_mask, uniform_sample):
    B, S = times.shape
    C = embeddings.shape[-1]
    T = wl.shape[-1]
    H = w1.shape[-1]
    K = min(7, S)
    N = int(uniform_sample.shape[0])
    NP1 = N + 1
    L = (S - 1) * NP1 + 1

    TM = 1024
    n_tiles = -(-L // TM)
    L_pad = TM * n_tiles
    SE = 224

    f32 = jnp.float32
    bf16 = jnp.bfloat16

    mx = jnp.max(times, axis=1)
    mx = jnp.where(mx > 0, mx, 1.0)
    ts_s = times / mx[:, None]

    ts_next = jnp.concatenate([ts_s[:, 1:], ts_s[:, -1:]], axis=1)
    tn = jnp.concatenate([times[:, 1:], times[:, -1:]], axis=1)
    npm_next = jnp.concatenate(
        [non_pad_mask[:, 1:], non_pad_mask[:, -1:]], axis=1)
    ev_idx = jnp.where(events - 1 == -1, 0, events - 1).astype(f32)
    zeros = jnp.zeros((B, S), f32)
    v_full = jnp.stack(
        [ts_s, ts_next, non_pad_mask, npm_next, tn - times, ev_idx,
         zeros, zeros], axis=1)

    starts = np.minimum(np.maximum(np.arange(n_tiles) * TM // NP1 - (K - 1),
                                   0), S - SE)
    starts = (starts // 8) * 8
    t_slab = jnp.stack(
        [jax.lax.slice(ts_s, (0, int(st)), (B, int(st) + SE))
         for st in starts], axis=1)[:, :, None, :]
    v_slab = jnp.stack(
        [jax.lax.slice(v_full, (0, 0, int(st)), (B, 8, int(st) + SE))
         for st in starts], axis=1)

    w2e = jnp.concatenate([w2, b2], axis=0)
    w2r = (w2e.reshape(H + 1, C, C).transpose(1, 0, 2)
           .reshape(C, (H + 1) * C).astype(bf16))
    wl_bf = wl.astype(bf16)
    sc = jnp.exp(softplus_params)[0]
    w1p = jnp.zeros((1, 128), f32).at[0, :H].set(w1[0])
    b1p = jnp.zeros((1, 128), f32).at[0, :H].set(b1[0])
    up = jnp.zeros((1, 128), f32).at[0, 1:NP1].set(uniform_sample)

    body = functools.partial(_fused_head_kernel, H=H, K=K, TM=TM, SE=SE,
                             NP1=NP1, S=S, L=L, T=T)
    out = pl.pallas_call(
        body,
        out_shape=jax.ShapeDtypeStruct((B, 1, 128), f32),
        grid=(B, n_tiles),
        in_specs=[
            pl.BlockSpec((None, None, 1, SE),
                         lambda b, i: (b, i, 0, 0)),
            pl.BlockSpec((None, None, 8, SE),
                         lambda b, i: (b, i, 0, 0)),
            pl.BlockSpec((None, S, C), lambda b, i: (b, 0, 0)),
            pl.BlockSpec((1, 128), lambda b, i: (0, 0)),
            pl.BlockSpec((1, 128), lambda b, i: (0, 0)),
            pl.BlockSpec((1, 128), lambda b, i: (0, 0)),
            pl.BlockSpec((C, (H + 1) * C), lambda b, i: (0, 0)),
            pl.BlockSpec((C, T), lambda b, i: (0, 0)),
            pl.BlockSpec((1, T), lambda b, i: (0, 0)),
            pl.BlockSpec((1, T), lambda b, i: (0, 0)),
        ],
        out_specs=pl.BlockSpec((None, 1, 128), lambda b, i: (b, 0, 0)),
        compiler_params=pltpu.CompilerParams(
            dimension_semantics=("parallel", "arbitrary")),
    )(t_slab, v_slab, embeddings, w1p, b1p, up, w2r, wl_bf, bl, sc)

    return jnp.sum(out[:, 0, 0])
